# baseline (speedup 1.0000x reference)
import jax
import jax.numpy as jnp
from jax import lax
from jax.experimental import pallas as pl
from jax.experimental.pallas import tpu as pltpu
from jax.experimental.pallas import tpu_sc as plsc

_N = 320000
_G = 10000
_GP = 10240
_NC = 2
_NS = 16
_NW = _NC * _NS
_R = _N // _NW
_CH = 128
_NCH = _R // _CH
_TAIL = _R - _NCH * _CH
_D = 4
_STRIPE = _GP // _NS
_SENT = _GP
_ACC = _GP + 16


def _sc_body(xflat, batch, part_out,
             ids_v, idx_v, col_v, tcol_v, acc_v, acc_n, shr_v, shr_n, red_v,
             sem0, sem1, sem2, sem3, semt, semi):
    sems = (sem0, sem1, sem2, sem3)
    cid = lax.axis_index("c")
    sid = lax.axis_index("s")
    base = (cid * _NS + sid) * _R

    ids_cp = pltpu.make_async_copy(batch.at[pl.ds(base, _R)], ids_v.at[pl.ds(0, _R)], semi)
    ids_cp.start()

    iota = lax.iota(jnp.int32, 16)
    for k in range(_CH // 16):
        idx_v[pl.ds(k * 16, 16)] = (iota + k * 16) * 128

    def _chunk_src(c):
        return xflat.at[pl.ds((base + c * _CH) * 128, _CH * 128)].at[idx_v]

    tail_cp = pltpu.make_async_copy(
        xflat.at[pl.ds((base + _NCH * _CH) * 128, _TAIL * 128)].at[idx_v.at[pl.ds(0, _TAIL)]],
        tcol_v, semt)
    tail_cp.start()

    zero = jnp.zeros((16,), jnp.float32)

    def _zero(j, carry):
        acc_v[pl.ds(j * 16, 16)] = zero
        acc_n[pl.ds(j * 16, 16)] = zero
        return carry

    lax.fori_loop(0, _ACC // 16, _zero, 0, unroll=8)

    def _chunk_copy(c, slot):
        return pltpu.make_async_copy(
            _chunk_src(c), col_v.at[pl.ds(slot * _CH, _CH)], sems[slot])

    for s in range(_D):
        _chunk_copy(s, s).start()

    ids_cp.wait()
    ids_v[pl.ds(_R, 16)] = jnp.full((16,), _SENT, jnp.int32)

    def _accum16(pos_ids, pos_col, vals_ref, tot):
        ids = ids_v[pl.ds(pos_ids, 16)]
        idsn = ids_v[pl.ds(pos_ids + 1, 16)]
        vals = vals_ref[pl.ds(pos_col, 16)]
        c = plsc.cumsum(vals) + tot
        m = ids != idsn
        plsc.addupdate_scatter(acc_v, [ids], c, mask=m)
        plsc.addupdate_scatter(acc_v, [idsn], -c, mask=m)
        p1 = (iota + (pos_ids + 1)).astype(jnp.float32)
        plsc.addupdate_scatter(acc_n, [ids], p1, mask=m)
        plsc.addupdate_scatter(acc_n, [idsn], -p1, mask=m)
        return tot + jnp.sum(vals)

    def _main(c, tot):
        slot = c % _D
        for s in range(_D):
            @pl.when(slot == s)
            def _w():
                _chunk_copy(c, s).wait()
        for k in range(_CH // 16):
            tot = _accum16(c * _CH + k * 16, slot * _CH + k * 16, col_v, tot)
        for s in range(_D):
            @pl.when((slot == s) & (c + _D < _NCH))
            def _f():
                _chunk_copy(c + _D, s).start()
        return tot

    tot = lax.fori_loop(0, _NCH, _main, jnp.float32(0.0))

    tail_cp.wait()
    _accum16(_NCH * _CH, 0, tcol_v, tot)

    pltpu.sync_copy(acc_v.at[pl.ds(0, _GP)], shr_v.at[sid])
    pltpu.sync_copy(acc_n.at[pl.ds(0, _GP)], shr_n.at[sid])
    plsc.subcore_barrier()

    off = sid * _STRIPE

    def _stripe_reduce(shr, acc):
        pltpu.sync_copy(shr.at[:, pl.ds(off, _STRIPE)], red_v)

        def _red(g, carry):
            s = red_v[0, pl.ds(g * 16, 16)]
            for r in range(1, _NS):
                s = s + red_v[r, pl.ds(g * 16, 16)]
            acc[pl.ds(g * 16, 16)] = s
            return carry

        lax.fori_loop(0, _STRIPE // 16, _red, 0, unroll=4)

    _stripe_reduce(shr_v, acc_v)
    _stripe_reduce(shr_n, acc_n)
    out_off = cid * _GP + off
    pltpu.sync_copy(acc_v.at[pl.ds(0, _STRIPE)], part_out.at[pl.ds(out_off, _STRIPE)])
    pltpu.sync_copy(acc_n.at[pl.ds(0, _STRIPE)], part_out.at[pl.ds(2 * _GP + out_off, _STRIPE)])


_sc_call = pl.kernel(
    _sc_body,
    out_type=jax.ShapeDtypeStruct((2 * _NC * _GP,), jnp.float32),
    mesh=plsc.VectorSubcoreMesh(core_axis_name="c", subcore_axis_name="s"),
    compiler_params=pltpu.CompilerParams(needs_layout_passes=False,
                                         skip_device_barrier=True),
    scratch_types=[
        pltpu.VMEM((_R + 16,), jnp.int32),
        pltpu.VMEM((_CH,), jnp.int32),
        pltpu.VMEM((_D * _CH,), jnp.float32),
        pltpu.VMEM((_TAIL,), jnp.float32),
        pltpu.VMEM((_ACC,), jnp.float32),
        pltpu.VMEM((_ACC,), jnp.float32),
        pltpu.VMEM_SHARED((_NS, _GP), jnp.float32),
        pltpu.VMEM_SHARED((_NS, _GP), jnp.float32),
        pltpu.VMEM((_NS, _STRIPE), jnp.float32),
        pltpu.SemaphoreType.DMA,
        pltpu.SemaphoreType.DMA,
        pltpu.SemaphoreType.DMA,
        pltpu.SemaphoreType.DMA,
        pltpu.SemaphoreType.DMA,
        pltpu.SemaphoreType.DMA,
    ],
)


def _merge_body(p_ref, o_ref):
    o_ref[...] = (p_ref[0] + p_ref[1]) / (p_ref[2] + p_ref[3])


def kernel(x, batch):
    xflat = x.reshape(-1)
    ids = batch.astype(jnp.int32)
    parts = _sc_call(xflat, ids)
    means = pl.pallas_call(
        _merge_body,
        out_shape=jax.ShapeDtypeStruct((_GP // 128, 128), jnp.float32),
    )(parts.reshape(2 * _NC, _GP // 128, 128))
    return means.reshape(_GP)[:_G][:, None]


# speedup vs baseline: 1.0031x; 1.0031x over previous
import jax
import jax.numpy as jnp
from jax import lax
from jax.experimental import pallas as pl
from jax.experimental.pallas import tpu as pltpu
from jax.experimental.pallas import tpu_sc as plsc

_N = 320000
_G = 10000
_GP = 10240
_NC = 2
_NS = 16
_NW = _NC * _NS
_R = _N // _NW
_CH = 128
_NCH = _R // _CH
_TAIL = _R - _NCH * _CH
_D = 4
_STRIPE = _GP // _NS
_SENT = _GP
_ACC = _GP + 16


def _sc_body(xflat, batch, part_out,
             ids_v, idx_v, col_v, tcol_v, acc_v, acc_n, shr_v, shr_n, red_v,
             sem0, sem1, sem2, sem3, semt, semi):
    sems = (sem0, sem1, sem2, sem3)
    cid = lax.axis_index("c")
    sid = lax.axis_index("s")
    base = (cid * _NS + sid) * _R

    ids_cp = pltpu.make_async_copy(batch.at[pl.ds(base, _R)], ids_v.at[pl.ds(0, _R)], semi)
    ids_cp.start()

    iota = lax.iota(jnp.int32, 16)
    for k in range(_CH // 16):
        idx_v[pl.ds(k * 16, 16)] = (iota + k * 16) * 128

    def _chunk_src(c):
        return xflat.at[pl.ds((base + c * _CH) * 128, _CH * 128)].at[idx_v]

    tail_cp = pltpu.make_async_copy(
        xflat.at[pl.ds((base + _NCH * _CH) * 128, _TAIL * 128)].at[idx_v.at[pl.ds(0, _TAIL)]],
        tcol_v, semt)
    tail_cp.start()

    zero = jnp.zeros((16,), jnp.float32)

    def _zero(j, carry):
        acc_v[pl.ds(j * 16, 16)] = zero
        acc_n[pl.ds(j * 16, 16)] = zero
        return carry

    lax.fori_loop(0, _ACC // 16, _zero, 0, unroll=8)

    def _chunk_copy(c, slot):
        return pltpu.make_async_copy(
            _chunk_src(c), col_v.at[pl.ds(slot * _CH, _CH)], sems[slot])

    for s in range(_D):
        _chunk_copy(s, s).start()

    ids_cp.wait()

    not_last = iota < 15
    is_last = iota >= 15
    p1c = (iota + 1).astype(jnp.float32)
    np1c = -p1c

    def _accum16(pos_ids, pos_col, vals_ref):
        ids = ids_v[pl.ds(pos_ids, 16)]
        idsn = ids_v[pl.ds(pos_ids + 1, 16)]
        vals = vals_ref[pl.ds(pos_col, 16)]
        c = plsc.cumsum(vals)
        m = ids != idsn
        mp = m | is_last
        mm = m & not_last
        plsc.addupdate_scatter(acc_v, [ids], c, mask=mp)
        plsc.addupdate_scatter(acc_v, [idsn], -c, mask=mm)
        plsc.addupdate_scatter(acc_n, [ids], p1c, mask=mp)
        plsc.addupdate_scatter(acc_n, [idsn], np1c, mask=mm)

    def _main(c, carry):
        slot = c % _D
        for s in range(_D):
            @pl.when(slot == s)
            def _w():
                _chunk_copy(c, s).wait()
        for k in range(_CH // 16):
            _accum16(c * _CH + k * 16, slot * _CH + k * 16, col_v)
        for s in range(_D):
            @pl.when((slot == s) & (c + _D < _NCH))
            def _f():
                _chunk_copy(c + _D, s).start()
        return carry

    lax.fori_loop(0, _NCH, _main, 0)

    tail_cp.wait()
    _accum16(_NCH * _CH, 0, tcol_v)

    pltpu.sync_copy(acc_v.at[pl.ds(0, _GP)], shr_v.at[sid])
    pltpu.sync_copy(acc_n.at[pl.ds(0, _GP)], shr_n.at[sid])
    plsc.subcore_barrier()

    off = sid * _STRIPE

    def _stripe_reduce(shr, acc):
        pltpu.sync_copy(shr.at[:, pl.ds(off, _STRIPE)], red_v)

        def _red(g, carry):
            s = red_v[0, pl.ds(g * 16, 16)]
            for r in range(1, _NS):
                s = s + red_v[r, pl.ds(g * 16, 16)]
            acc[pl.ds(g * 16, 16)] = s
            return carry

        lax.fori_loop(0, _STRIPE // 16, _red, 0, unroll=4)

    _stripe_reduce(shr_v, acc_v)
    _stripe_reduce(shr_n, acc_n)
    out_off = cid * _GP + off
    pltpu.sync_copy(acc_v.at[pl.ds(0, _STRIPE)], part_out.at[pl.ds(out_off, _STRIPE)])
    pltpu.sync_copy(acc_n.at[pl.ds(0, _STRIPE)], part_out.at[pl.ds(2 * _GP + out_off, _STRIPE)])


_sc_call = pl.kernel(
    _sc_body,
    out_type=jax.ShapeDtypeStruct((2 * _NC * _GP,), jnp.float32),
    mesh=plsc.VectorSubcoreMesh(core_axis_name="c", subcore_axis_name="s"),
    compiler_params=pltpu.CompilerParams(needs_layout_passes=False,
                                         skip_device_barrier=True),
    scratch_types=[
        pltpu.VMEM((_R + 16,), jnp.int32),
        pltpu.VMEM((_CH,), jnp.int32),
        pltpu.VMEM((_D * _CH,), jnp.float32),
        pltpu.VMEM((_TAIL,), jnp.float32),
        pltpu.VMEM((_ACC,), jnp.float32),
        pltpu.VMEM((_ACC,), jnp.float32),
        pltpu.VMEM_SHARED((_NS, _GP), jnp.float32),
        pltpu.VMEM_SHARED((_NS, _GP), jnp.float32),
        pltpu.VMEM((_NS, _STRIPE), jnp.float32),
        pltpu.SemaphoreType.DMA,
        pltpu.SemaphoreType.DMA,
        pltpu.SemaphoreType.DMA,
        pltpu.SemaphoreType.DMA,
        pltpu.SemaphoreType.DMA,
        pltpu.SemaphoreType.DMA,
    ],
)


def _merge_body(p_ref, o_ref):
    o_ref[...] = (p_ref[0] + p_ref[1]) / (p_ref[2] + p_ref[3])


def kernel(x, batch):
    xflat = x.reshape(-1)
    ids = batch.astype(jnp.int32)
    parts = _sc_call(xflat, ids)
    means = pl.pallas_call(
        _merge_body,
        out_shape=jax.ShapeDtypeStruct((_GP // 128, 128), jnp.float32),
    )(parts.reshape(2 * _NC, _GP // 128, 128))
    return means.reshape(_GP)[:_G][:, None]


# speedup vs baseline: 1.0806x; 1.0773x over previous
import jax
import jax.numpy as jnp
from jax import lax
from jax.experimental import pallas as pl
from jax.experimental.pallas import tpu as pltpu
from jax.experimental.pallas import tpu_sc as plsc

_N = 320000
_G = 10000
_GP = 10240
_NC = 2
_NS = 16
_NW = _NC * _NS
_R = _N // _NW
_CH = 128
_NCH = _R // _CH
_TAIL = _R - _NCH * _CH
_D = 6
_STRIPE = _GP // _NS
_SENT = _GP
_ACC = _GP + 16


def _sc_body(xflat, batch, part_out,
             ids_v, idx_v, col_v, tcol_v, acc_v, acc_n, shr_v, shr_n, red_v,
             sem0, sem1, sem2, sem3, sem4, sem5, semt, semi):
    sems = (sem0, sem1, sem2, sem3, sem4, sem5)
    cid = lax.axis_index("c")
    sid = lax.axis_index("s")
    base = (cid * _NS + sid) * _R

    ids_cp = pltpu.make_async_copy(batch.at[pl.ds(base, _R)], ids_v.at[pl.ds(0, _R)], semi)
    ids_cp.start()

    iota = lax.iota(jnp.int32, 16)
    for k in range(_CH // 16):
        idx_v[pl.ds(k * 16, 16)] = (iota + k * 16) * 128

    def _chunk_src(c):
        return xflat.at[pl.ds((base + c * _CH) * 128, _CH * 128)].at[idx_v]

    tail_cp = pltpu.make_async_copy(
        xflat.at[pl.ds((base + _NCH * _CH) * 128, _TAIL * 128)].at[idx_v.at[pl.ds(0, _TAIL)]],
        tcol_v, semt)
    tail_cp.start()

    zero = jnp.zeros((16,), jnp.float32)

    def _zero(j, carry):
        acc_v[pl.ds(j * 16, 16)] = zero
        acc_n[pl.ds(j * 16, 16)] = zero
        return carry

    lax.fori_loop(0, _ACC // 16, _zero, 0, unroll=8)

    def _chunk_copy(c, slot):
        return pltpu.make_async_copy(
            _chunk_src(c), col_v.at[pl.ds(slot * _CH, _CH)], sems[slot])

    for s in range(_D):
        _chunk_copy(s, s).start()

    ids_cp.wait()

    not_last = iota < 15
    is_last = iota >= 15
    p1c = (iota + 1).astype(jnp.float32)
    np1c = -p1c

    def _accum16(pos_ids, pos_col, vals_ref):
        ids = ids_v[pl.ds(pos_ids, 16)]
        idsn = ids_v[pl.ds(pos_ids + 1, 16)]
        vals = vals_ref[pl.ds(pos_col, 16)]
        c = plsc.cumsum(vals)
        m = ids != idsn
        mp = m | is_last
        mm = m & not_last
        plsc.addupdate_scatter(acc_v, [ids], c, mask=mp)
        plsc.addupdate_scatter(acc_v, [idsn], -c, mask=mm)
        plsc.addupdate_scatter(acc_n, [ids], p1c, mask=mp)
        plsc.addupdate_scatter(acc_n, [idsn], np1c, mask=mm)

    def _round(i, carry):
        c0 = i * _D
        for s in range(_D):
            c = c0 + s
            _chunk_copy(c, s).wait()
            for k in range(_CH // 16):
                _accum16(c * _CH + k * 16, s * _CH + k * 16, col_v)

            @pl.when(c + _D < _NCH)
            def _f():
                _chunk_copy(c + _D, s).start()
        return carry

    lax.fori_loop(0, _NCH // _D, _round, 0)

    tail_cp.wait()
    _accum16(_NCH * _CH, 0, tcol_v)

    pltpu.sync_copy(acc_v.at[pl.ds(0, _GP)], shr_v.at[sid])
    pltpu.sync_copy(acc_n.at[pl.ds(0, _GP)], shr_n.at[sid])
    plsc.subcore_barrier()

    off = sid * _STRIPE

    def _stripe_reduce(shr, acc):
        pltpu.sync_copy(shr.at[:, pl.ds(off, _STRIPE)], red_v)

        def _red(g, carry):
            s = red_v[0, pl.ds(g * 16, 16)]
            for r in range(1, _NS):
                s = s + red_v[r, pl.ds(g * 16, 16)]
            acc[pl.ds(g * 16, 16)] = s
            return carry

        lax.fori_loop(0, _STRIPE // 16, _red, 0, unroll=4)

    _stripe_reduce(shr_v, acc_v)
    _stripe_reduce(shr_n, acc_n)
    out_off = cid * _GP + off
    pltpu.sync_copy(acc_v.at[pl.ds(0, _STRIPE)], part_out.at[pl.ds(out_off, _STRIPE)])
    pltpu.sync_copy(acc_n.at[pl.ds(0, _STRIPE)], part_out.at[pl.ds(2 * _GP + out_off, _STRIPE)])


_sc_call = pl.kernel(
    _sc_body,
    out_type=jax.ShapeDtypeStruct((2 * _NC * _GP,), jnp.float32),
    mesh=plsc.VectorSubcoreMesh(core_axis_name="c", subcore_axis_name="s"),
    compiler_params=pltpu.CompilerParams(needs_layout_passes=False,
                                         skip_device_barrier=True),
    scratch_types=[
        pltpu.VMEM((_R + 16,), jnp.int32),
        pltpu.VMEM((_CH,), jnp.int32),
        pltpu.VMEM((_D * _CH,), jnp.float32),
        pltpu.VMEM((_TAIL,), jnp.float32),
        pltpu.VMEM((_ACC,), jnp.float32),
        pltpu.VMEM((_ACC,), jnp.float32),
        pltpu.VMEM_SHARED((_NS, _GP), jnp.float32),
        pltpu.VMEM_SHARED((_NS, _GP), jnp.float32),
        pltpu.VMEM((_NS, _STRIPE), jnp.float32),
        pltpu.SemaphoreType.DMA,
        pltpu.SemaphoreType.DMA,
        pltpu.SemaphoreType.DMA,
        pltpu.SemaphoreType.DMA,
        pltpu.SemaphoreType.DMA,
        pltpu.SemaphoreType.DMA,
        pltpu.SemaphoreType.DMA,
        pltpu.SemaphoreType.DMA,
    ],
)


def _merge_body(p_ref, o_ref):
    o_ref[...] = (p_ref[0] + p_ref[1]) / (p_ref[2] + p_ref[3])


def kernel(x, batch):
    xflat = x.reshape(-1)
    ids = batch.astype(jnp.int32)
    parts = _sc_call(xflat, ids)
    means = pl.pallas_call(
        _merge_body,
        out_shape=jax.ShapeDtypeStruct((_GP // 128, 128), jnp.float32),
    )(parts.reshape(2 * _NC, _GP // 128, 128))
    return means.reshape(_GP)[:_G][:, None]
